# baseline (speedup 1.0000x reference)
import functools

import jax
import jax.numpy as jnp
from jax import lax
from jax.experimental import pallas as pl
from jax.experimental.pallas import tpu as pltpu
from jax.experimental.pallas import tpu_sc as plsc

N_ROWS = 1000000
F = 32
B = 16384
H = 8
_TILES = N_ROWS // 8

_NC = 2
_NS = 16
_NW = _NC * _NS
_BPW = B // _NW
_CH = 32
_NCHUNK = _BPW // _CH
_L = 16


def _extract_rows(buf, idx_v, stage, off):
    lanes = lax.iota(jnp.int32, _L)
    for g in range(_CH // _L):
        sub = jnp.bitwise_and(idx_v[pl.ds(off + g * _L, _L)], 7)
        li = lanes + g * _L
        for c in range(F):
            cc = jnp.full((_L,), c, jnp.int32)
            vals = plsc.load_gather(buf, [li, sub, cc])
            plsc.store_scatter(stage, [li, cc], vals)


def _gather_body(user_hbm, item_hbm, uf_hbm, if_hbm, ue_out, ie_out,
                 uidx_v, iidx_v, ubuf, ibuf, ustage, istage, sem, sem_i):
    wid = lax.axis_index("s") * _NC + lax.axis_index("c")
    base = wid * _BPW
    pltpu.sync_copy(user_hbm.at[pl.ds(base, _BPW)], uidx_v)
    pltpu.sync_copy(item_hbm.at[pl.ds(base, _BPW)], iidx_v)

    def chunk_body(k, _):
        off = k * _CH
        ucopies, icopies = [], []
        for g in range(_CH // _L):
            ut_vec = jnp.right_shift(uidx_v[pl.ds(off + g * _L, _L)], 3)
            it_vec = jnp.right_shift(iidx_v[pl.ds(off + g * _L, _L)], 3)
            for r in range(_L):
                row = g * _L + r
                ucopies.append(
                    pltpu.async_copy(uf_hbm.at[ut_vec[r]], ubuf.at[row], sem))
                icopies.append(
                    pltpu.async_copy(if_hbm.at[it_vec[r]], ibuf.at[row], sem_i))
        for cp in ucopies:
            cp.wait()
        for cp in icopies:
            cp.wait()
        _extract_rows(ubuf, uidx_v, ustage, off)
        _extract_rows(ibuf, iidx_v, istage, off)
        pltpu.sync_copy(ustage, ue_out.at[pl.ds(base + off, _CH)])
        pltpu.sync_copy(istage, ie_out.at[pl.ds(base + off, _CH)])
        return ()

    lax.fori_loop(0, _NCHUNK, chunk_body, (), unroll=False)


_sc_gather = functools.partial(
    pl.kernel,
    out_type=[
        jax.ShapeDtypeStruct((B, 128), jnp.float32),
        jax.ShapeDtypeStruct((B, 128), jnp.float32),
    ],
    mesh=plsc.VectorSubcoreMesh(core_axis_name="c", subcore_axis_name="s"),
    scratch_types=[
        pltpu.VMEM((_BPW,), jnp.int32),
        pltpu.VMEM((_BPW,), jnp.int32),
        pltpu.VMEM((_CH, 8, F), jnp.float32),
        pltpu.VMEM((_CH, 8, F), jnp.float32),
        pltpu.VMEM((_CH, 128), jnp.float32),
        pltpu.VMEM((_CH, 128), jnp.float32),
        pltpu.SemaphoreType.DMA,
        pltpu.SemaphoreType.DMA,
    ],
    compiler_params=pltpu.CompilerParams(needs_layout_passes=False),
)(_gather_body)


def _mlp_body(ue_ref, ie_ref, w1u_ref, w1i_ref, b1_ref, w3_ref, b3_ref, out_ref):
    ue = ue_ref[...][:, :F]
    ie = ie_ref[...][:, :F]
    h = (jnp.dot(ue, w1u_ref[...], preferred_element_type=jnp.float32)
         + jnp.dot(ie, w1i_ref[...], preferred_element_type=jnp.float32)
         + b1_ref[...])
    h = jnp.maximum(h, 0.0)
    z = jnp.dot(h, w3_ref[...], preferred_element_type=jnp.float32) + b3_ref[...]
    out_ref[...] = jax.nn.sigmoid(z)


_BLK = 4096


def _mlp(ue, ie, w1u, w1i, b1, w3, b3):
    grid = (B // _BLK,)
    return pl.pallas_call(
        _mlp_body,
        grid=grid,
        in_specs=[
            pl.BlockSpec((_BLK, 128), lambda i: (i, 0)),
            pl.BlockSpec((_BLK, 128), lambda i: (i, 0)),
            pl.BlockSpec((F, H), lambda i: (0, 0)),
            pl.BlockSpec((F, H), lambda i: (0, 0)),
            pl.BlockSpec((1, H), lambda i: (0, 0)),
            pl.BlockSpec((H, 1), lambda i: (0, 0)),
            pl.BlockSpec((1, 1), lambda i: (0, 0)),
        ],
        out_specs=pl.BlockSpec((_BLK, 1), lambda i: (i, 0)),
        out_shape=jax.ShapeDtypeStruct((B, 1), jnp.float32),
    )(ue, ie, w1u, w1i, b1, w3, b3)


def kernel(user, item, user_factors, item_factors, W1, b1, W3, b3):
    user = user.astype(jnp.int32)
    item = item.astype(jnp.int32)
    uf3 = user_factors.reshape(_TILES, 8, F)
    if3 = item_factors.reshape(_TILES, 8, F)
    ue, ie = _sc_gather(user, item, uf3, if3)
    return _mlp(ue, ie, W1[:F], W1[F:], b1.reshape(1, H), W3, b3.reshape(1, 1))


# speedup vs baseline: 1.0113x; 1.0113x over previous
import functools

import jax
import jax.numpy as jnp
from jax import lax
from jax.experimental import pallas as pl
from jax.experimental.pallas import tpu as pltpu
from jax.experimental.pallas import tpu_sc as plsc

N_ROWS = 1000000
F = 32
B = 16384
H = 8
_TILES = N_ROWS // 8

_NC = 2
_NS = 16
_NW = _NC * _NS
_BPW = B // _NW
_CH = 32
_NCHUNK = _BPW // _CH
_L = 16


def _extract_rows(buf, idx_v, stage, off):
    lanes = lax.iota(jnp.int32, _L)
    for g in range(_CH // _L):
        sub = jnp.bitwise_and(idx_v[pl.ds(off + g * _L, _L)], 7)
        li = lanes + g * _L
        for c in range(F):
            cc = jnp.full((_L,), c, jnp.int32)
            vals = plsc.load_gather(buf, [li, sub, cc])
            plsc.store_scatter(stage, [li, cc], vals)


def _gather_body(user_hbm, item_hbm, uf_hbm, if_hbm, ue_out, ie_out,
                 uidx_v, iidx_v, ubuf, ibuf, ustage, istage, sem):
    wid = lax.axis_index("s") * _NC + lax.axis_index("c")
    base = wid * _BPW
    pltpu.sync_copy(user_hbm.at[pl.ds(base, _BPW)], uidx_v)
    pltpu.sync_copy(item_hbm.at[pl.ds(base, _BPW)], iidx_v)

    def chunk_body(k, _):
        off = k * _CH
        copies = []
        for g in range(_CH // _L):
            ut_vec = jnp.right_shift(uidx_v[pl.ds(off + g * _L, _L)], 3)
            it_vec = jnp.right_shift(iidx_v[pl.ds(off + g * _L, _L)], 3)
            for r in range(_L):
                row = g * _L + r
                copies.append(
                    pltpu.async_copy(uf_hbm.at[ut_vec[r]], ubuf.at[row], sem))
                copies.append(
                    pltpu.async_copy(if_hbm.at[it_vec[r]], ibuf.at[row], sem))
        for cp in copies:
            cp.wait()
        _extract_rows(ubuf, uidx_v, ustage, off)
        _extract_rows(ibuf, iidx_v, istage, off)
        pltpu.sync_copy(ustage, ue_out.at[pl.ds(base + off, _CH)])
        pltpu.sync_copy(istage, ie_out.at[pl.ds(base + off, _CH)])
        return ()

    lax.fori_loop(0, _NCHUNK, chunk_body, (), unroll=False)


_sc_gather = functools.partial(
    pl.kernel,
    out_type=[
        jax.ShapeDtypeStruct((B, F), jnp.float32),
        jax.ShapeDtypeStruct((B, F), jnp.float32),
    ],
    mesh=plsc.VectorSubcoreMesh(core_axis_name="c", subcore_axis_name="s"),
    scratch_types=[
        pltpu.VMEM((_BPW,), jnp.int32),
        pltpu.VMEM((_BPW,), jnp.int32),
        pltpu.VMEM((_CH, 8, F), jnp.float32),
        pltpu.VMEM((_CH, 8, F), jnp.float32),
        pltpu.VMEM((_CH, F), jnp.float32),
        pltpu.VMEM((_CH, F), jnp.float32),
        pltpu.SemaphoreType.DMA,
    ],
    compiler_params=pltpu.CompilerParams(needs_layout_passes=False),
)(_gather_body)


def _mlp_body(ue_ref, ie_ref, w1u_ref, w1i_ref, b1_ref, w3_ref, b3_ref, out_ref):
    h = (jnp.dot(ue_ref[...], w1u_ref[...], preferred_element_type=jnp.float32)
         + jnp.dot(ie_ref[...], w1i_ref[...], preferred_element_type=jnp.float32)
         + b1_ref[...])
    h = jnp.maximum(h, 0.0)
    z = jnp.dot(h, w3_ref[...], preferred_element_type=jnp.float32) + b3_ref[...]
    out_ref[...] = jax.nn.sigmoid(z)


_BLK = 4096


def _mlp(ue, ie, w1u, w1i, b1, w3, b3):
    grid = (B // _BLK,)
    return pl.pallas_call(
        _mlp_body,
        grid=grid,
        in_specs=[
            pl.BlockSpec((_BLK, F), lambda i: (i, 0)),
            pl.BlockSpec((_BLK, F), lambda i: (i, 0)),
            pl.BlockSpec((F, H), lambda i: (0, 0)),
            pl.BlockSpec((F, H), lambda i: (0, 0)),
            pl.BlockSpec((1, H), lambda i: (0, 0)),
            pl.BlockSpec((H, 1), lambda i: (0, 0)),
            pl.BlockSpec((1, 1), lambda i: (0, 0)),
        ],
        out_specs=pl.BlockSpec((_BLK, 1), lambda i: (i, 0)),
        out_shape=jax.ShapeDtypeStruct((B, 1), jnp.float32),
    )(ue, ie, w1u, w1i, b1, w3, b3)


def kernel(user, item, user_factors, item_factors, W1, b1, W3, b3):
    user = user.astype(jnp.int32)
    item = item.astype(jnp.int32)
    uf3 = user_factors.reshape(_TILES, 8, F)
    if3 = item_factors.reshape(_TILES, 8, F)
    ue, ie = _sc_gather(user, item, uf3, if3)
    return _mlp(ue, ie, W1[:F], W1[F:], b1.reshape(1, H), W3, b3.reshape(1, 1))
